# baseline (speedup 1.0000x reference)
import functools

import jax
import jax.numpy as jnp
from jax import lax
from jax.experimental import pallas as pl
from jax.experimental.pallas import tpu as pltpu
from jax.experimental.pallas import tpu_sc as plsc

B = 16384
S = 50
D = 64
NC, NS = 2, 16
NW = NC * NS
GC = B // NW
C = 256
NB1 = C // 128

_mesh = plsc.VectorSubcoreMesh(core_axis_name="c", subcore_axis_name="s")


@functools.partial(
    pl.kernel,
    mesh=_mesh,
    out_type=jax.ShapeDtypeStruct((S, 8, B // 128, 8, 128), jnp.float32),
    compiler_params=pltpu.CompilerParams(use_tc_tiling_on_sc=False, needs_layout_passes=False),
    scratch_types=[
        pltpu.VMEM((S, GC), jnp.int32),
        pltpu.VMEM((C, D), jnp.float32),
        pltpu.VMEM((C, D), jnp.float32),
        pltpu.VMEM((8, NB1, 8, 128), jnp.float32),
        pltpu.VMEM((8, NB1, 8, 128), jnp.float32),
        pltpu.SemaphoreType.DMA,
        pltpu.SemaphoreType.DMA,
        pltpu.SemaphoreType.DMA,
        pltpu.SemaphoreType.DMA,
    ],
)
def _emb_lookup(xt_hbm, table_hbm, out_hbm, idx_v, rows_0, rows_1,
                tblk_0, tblk_1, g0, g1, w0, w1):
    rows_v = (rows_0, rows_1)
    tblk_v = (tblk_0, tblk_1)
    wid = lax.axis_index("s") * NC + lax.axis_index("c")
    col = wid * GC
    pltpu.sync_copy(xt_hbm.at[:, pl.ds(col, GC)], idx_v)

    gsem = (g0, g1)
    wsem = (w0, w1)

    def gather(s, h, b):
        return pltpu.make_async_copy(
            table_hbm.at[idx_v.at[s, pl.ds(h * C, C)]], rows_v[b], gsem[b])

    def write(s, h, b):
        return pltpu.make_async_copy(
            tblk_v[b],
            out_hbm.at[s, :, pl.ds(wid * 4 + h * NB1, NB1), :, :],
            wsem[b])

    riota = lax.iota(jnp.int32, 16)

    def transpose(b):
        rows = rows_v[b]
        blk = tblk_v[b]
        for d1 in range(8):
            for b1 in range(NB1):
                for b2g in range(8):
                    row_idx = riota + (b1 * 128 + b2g * 16)
                    for d2 in range(8):
                        d = d1 * 8 + d2
                        col_idx = riota * 0 + d
                        v = plsc.load_gather(rows, [row_idx, col_idx])
                        blk[d1, b1, d2, pl.ds(b2g * 16, 16)] = v

    gather(0, 0, 0).start()

    def outer(tt, carry):
        for b in range(2):
            t = 2 * tt + b
            s, h = tt, b
            @pl.when(t + 1 < 2 * S)
            def _():
                s1, h1 = (t + 1) // 2, (t + 1) % 2
                gather(s1, h1, 1 - b).start()

            gather(s, h, b).wait()

            @pl.when(tt >= 1)
            def _():
                write(tt - 1, h, b).wait()

            transpose(b)
            write(s, h, b).start()
        return carry

    lax.fori_loop(0, S, outer, 0)
    write(S - 1, 0, 0).wait()
    write(S - 1, 1, 1).wait()


def kernel(x, table):
    out = _emb_lookup(x.T, table)
    return out.transpose(2, 4, 0, 1, 3).reshape(B, S, D)


# speedup vs baseline: 1.2674x; 1.2674x over previous
import functools

import jax
import jax.numpy as jnp
from jax import lax
from jax.experimental import pallas as pl
from jax.experimental.pallas import tpu as pltpu
from jax.experimental.pallas import tpu_sc as plsc

B = 16384
S = 50
D = 64
NC, NS = 2, 16
NW = NC * NS
GC = B // NW
C = 256
NB1 = C // 128
TB = 8 * NB1 * 8 * 128

_mesh = plsc.VectorSubcoreMesh(core_axis_name="c", subcore_axis_name="s")


@functools.partial(
    pl.kernel,
    mesh=_mesh,
    out_type=jax.ShapeDtypeStruct((S * 8 * (B // 128) * 8 * 128,), jnp.float32),
    compiler_params=pltpu.CompilerParams(
        use_tc_tiling_on_sc=False, needs_layout_passes=False),
    scratch_types=[
        pltpu.VMEM((S, GC), jnp.int32),
        pltpu.VMEM((C, D), jnp.float32),
        pltpu.VMEM((C, D), jnp.float32),
        pltpu.VMEM((TB,), jnp.float32),
        pltpu.VMEM((TB,), jnp.float32),
        pltpu.SemaphoreType.DMA,
        pltpu.SemaphoreType.DMA,
        pltpu.SemaphoreType.DMA,
        pltpu.SemaphoreType.DMA,
    ],
)
def _emb_lookup(xt_hbm, table_hbm, out_hbm, idx_v, rows_0, rows_1,
                tblk_0, tblk_1, g0, g1, w0, w1):
    rows_v = (rows_0, rows_1)
    tblk_v = (tblk_0, tblk_1)
    wid = lax.axis_index("s") * NC + lax.axis_index("c")
    col = wid * GC
    pltpu.sync_copy(xt_hbm.at[:, pl.ds(col, GC)], idx_v)

    gsem = (g0, g1)
    wsem = (w0, w1)

    def gather(s, h, b):
        return pltpu.make_async_copy(
            table_hbm.at[idx_v.at[s, pl.ds(h * C, C)]], rows_v[b], gsem[b])

    def writes(s, h, b):
        cps = []
        for d1 in range(8):
            base = ((s * 8 + d1) * (B // 128) + (wid * 4 + h * NB1)) * 1024
            cps.append(pltpu.make_async_copy(
                tblk_v[b].at[pl.ds(d1 * NB1 * 1024, NB1 * 1024)],
                out_hbm.at[pl.ds(base, NB1 * 1024)],
                wsem[b]))
        return cps

    riota = lax.iota(jnp.int32, 16)
    pat = ((riota >> 3) << 11) + ((riota & 7) << 7)

    def transpose(b):
        rows = rows_v[b]
        blk = tblk_v[b]

        def body(rr, carry):
            r0 = rr * 8
            for dr in range(8):
                r = r0 + dr
                base = (r >> 7) * 1024 + (r & 127)
                for g in range(4):
                    v = rows[r, pl.ds(g * 16, 16)]
                    plsc.store_scatter(blk, [pat + (base + g * 4096)], v)
            return carry

        lax.fori_loop(0, C // 8, body, 0)

    gather(0, 0, 0).start()

    def outer(tt, carry):
        for b in range(2):
            t = 2 * tt + b
            s, h = tt, b
            @pl.when(t + 1 < 2 * S)
            def _():
                s1, h1 = (t + 1) // 2, (t + 1) % 2
                gather(s1, h1, 1 - b).start()

            gather(s, h, b).wait()

            @pl.when(tt >= 1)
            def _():
                for cp in writes(tt - 1, h, b):
                    cp.wait()

            transpose(b)
            for cp in writes(s, h, b):
                cp.start()
        return carry

    lax.fori_loop(0, S, outer, 0)
    for cp in writes(S - 1, 0, 0):
        cp.wait()
    for cp in writes(S - 1, 1, 1):
        cp.wait()


def kernel(x, table):
    out = _emb_lookup(x.T, table)
    out = out.reshape(S, 8, B // 128, 8, 128)
    return out.transpose(2, 4, 0, 1, 3).reshape(B, S, D)


# speedup vs baseline: 1.3039x; 1.0288x over previous
import functools

import jax
import jax.numpy as jnp
from jax import lax
from jax.experimental import pallas as pl
from jax.experimental.pallas import tpu as pltpu
from jax.experimental.pallas import tpu_sc as plsc

B = 16384
S = 50
D = 64
NC, NS = 2, 16
NW = NC * NS
GC = B // NW
C = 256
NB1 = C // 128
TB = 8 * NB1 * 8 * 128

_mesh = plsc.VectorSubcoreMesh(core_axis_name="c", subcore_axis_name="s")


@functools.partial(
    pl.kernel,
    mesh=_mesh,
    out_type=jax.ShapeDtypeStruct((S * 8 * (B // 128) * 8 * 128,), jnp.float32),
    compiler_params=pltpu.CompilerParams(
        use_tc_tiling_on_sc=False, needs_layout_passes=False),
    scratch_types=[
        pltpu.VMEM((S, GC), jnp.int32),
        pltpu.VMEM((C, D), jnp.float32),
        pltpu.VMEM((C, D), jnp.float32),
        pltpu.VMEM((TB,), jnp.float32),
        pltpu.VMEM((TB,), jnp.float32),
        pltpu.SemaphoreType.DMA,
        pltpu.SemaphoreType.DMA,
        pltpu.SemaphoreType.DMA,
        pltpu.SemaphoreType.DMA,
    ],
)
def _emb_lookup(xt_hbm, table_hbm, out_hbm, idx_v, rows_0, rows_1,
                tblk_0, tblk_1, g0, g1, w0, w1):
    rows_v = (rows_0, rows_1)
    tblk_v = (tblk_0, tblk_1)
    wid = lax.axis_index("s") * NC + lax.axis_index("c")
    col = wid * GC
    pltpu.sync_copy(xt_hbm.at[:, pl.ds(col, GC)], idx_v)

    gsem = (g0, g1)
    wsem = (w0, w1)

    def gather(s, h, b):
        return pltpu.make_async_copy(
            table_hbm.at[idx_v.at[s, pl.ds(h * C, C)]], rows_v[b], gsem[b])

    def writes(s, h, b):
        cps = []
        for d1 in range(8):
            base = ((s * 8 + d1) * (B // 128) + (wid * 4 + h * NB1)) * 1024
            cps.append(pltpu.make_async_copy(
                tblk_v[b].at[pl.ds(d1 * NB1 * 1024, NB1 * 1024)],
                out_hbm.at[pl.ds(base, NB1 * 1024)],
                wsem[b]))
        return cps

    riota = lax.iota(jnp.int32, 16)
    pat = ((riota >> 3) << 11) + ((riota & 7) << 7)

    def transpose(b):
        rows = rows_v[b]
        blk = tblk_v[b]

        def body(rr, carry):
            r0 = rr * 8
            for half in range(2):
                vs, idxs = [], []
                for dr in range(4):
                    r = r0 + half * 4 + dr
                    base = (r >> 7) * 1024 + (r & 127)
                    for g in range(4):
                        vs.append(rows[r, pl.ds(g * 16, 16)])
                        idxs.append(pat + (base + g * 4096))
                for v, ix in zip(vs, idxs):
                    plsc.store_scatter(blk, [ix], v)
            return carry

        lax.fori_loop(0, C // 8, body, 0)

    gather(0, 0, 0).start()

    def outer(tt, carry):
        for b in range(2):
            t = 2 * tt + b
            s, h = tt, b
            @pl.when(t + 1 < 2 * S)
            def _():
                s1, h1 = (t + 1) // 2, (t + 1) % 2
                gather(s1, h1, 1 - b).start()

            gather(s, h, b).wait()

            @pl.when(tt >= 1)
            def _():
                for cp in writes(tt - 1, h, b):
                    cp.wait()

            transpose(b)
            for cp in writes(s, h, b):
                cp.start()
        return carry

    lax.fori_loop(0, S, outer, 0)
    for cp in writes(S - 1, 0, 0):
        cp.wait()
    for cp in writes(S - 1, 1, 1):
        cp.wait()


def kernel(x, table):
    out = _emb_lookup(x.T, table)
    out = out.reshape(S, 8, B // 128, 8, 128)
    return out.transpose(2, 4, 0, 1, 3).reshape(B, S, D)
